# baseline (speedup 1.0000x reference)
import functools

import jax
import jax.numpy as jnp
from jax import lax
from jax.experimental import pallas as pl
from jax.experimental.pallas import tpu as pltpu
from jax.experimental.pallas import tpu_sc as plsc

_NC = 8
_N = 20000
_NV = _N // 16
_BLK = 16
_NB = (_NV + _BLK - 1) // _BLK
_NL1 = 80
_MD = 100
_SLOTS = 112
_NMS_T = 0.5
_SCORE_T = 0.01
_NEG = -1e9
_BIG = 2**30
_SENT = 2e9


def _lane():
    return lax.iota(jnp.int32, 16)


def _splat_f(x):
    return jnp.full((16,), x, dtype=jnp.float32)


def _splat_i(x):
    return jnp.full((16,), x, dtype=jnp.int32)


def _sc_body(scores_h, x1_h, y1_h, x2_h, y2_h,
             t0_h, t1_h, t2_h, r0_h, r1_h, r2_h,
             of_h, os_h, ol_h,
             sc_v, x1_v, y1_v, x2_v, y2_v, l1_v,
             ax1_v, ay1_v, ax2_v, ay2_v, osc_v, oidx_v,
             sh_sc, sh_idx, msc_v, midx_v, sout_v, lout_v, gidx_v, big_v,
             dsem, bsem):
    core = lax.axis_index("c")
    sub = lax.axis_index("s")
    lane = _lane()
    active = (core == 0) & (sub < _NC)

    @pl.when(active)
    def _per_class():
        bcopies = [
            pltpu.async_copy(x1_h, x1_v, bsem),
            pltpu.async_copy(y1_h, y1_v, bsem),
            pltpu.async_copy(x2_h, x2_v, bsem),
            pltpu.async_copy(y2_h, y2_v, bsem),
        ]
        pltpu.sync_copy(scores_h.at[pl.ds(sub * _N, _N)], sc_v)

        def build_block(g, _):
            nv = jnp.minimum(_BLK, _NV - g * _BLK)

            def inner(k, acc):
                i = g * _BLK + k
                v = sc_v[pl.ds(i * 16, 16)]
                v = jnp.where(v >= _SCORE_T, v, _NEG)
                sc_v[pl.ds(i * 16, 16)] = v
                return jnp.maximum(acc, v)

            acc = lax.fori_loop(0, nv, inner, _splat_f(_NEG))
            m = jnp.max(acc)
            jb = (g // 16) * 16
            l1_v[pl.ds(jb, 16)] = jnp.where(
                lane == (g - jb), m, l1_v[pl.ds(jb, 16)])
            return 0

        l1_v[pl.ds(4 * 16, 16)] = _splat_f(_NEG)
        lax.fori_loop(0, _NB, build_block, 0)

        def init_slots(t, _):
            ax1_v[pl.ds(t * 16, 16)] = _splat_f(_SENT)
            ay1_v[pl.ds(t * 16, 16)] = _splat_f(_SENT)
            ax2_v[pl.ds(t * 16, 16)] = _splat_f(_SENT)
            ay2_v[pl.ds(t * 16, 16)] = _splat_f(_SENT)
            osc_v[pl.ds(t * 16, 16)] = _splat_f(_NEG)
            oidx_v[pl.ds(t * 16, 16)] = _splat_i(-1)
            return 0

        lax.fori_loop(0, _SLOTS // 16, init_slots, 0)
        for cp in bcopies:
            cp.wait()

        def greedy_cond(state):
            count, done = state
            return (count < _MD) & jnp.logical_not(done)

        def greedy_body(state):
            count, done = state
            m1 = _splat_f(_NEG)
            for t in range(_NL1 // 16):
                m1 = jnp.maximum(m1, l1_v[pl.ds(t * 16, 16)])
            s = jnp.max(m1)
            valid = s > _NEG / 2
            g = _BIG
            for t in range(_NL1 // 16):
                v1 = l1_v[pl.ds(t * 16, 16)]
                g = jnp.minimum(
                    g, jnp.min(jnp.where(v1 == s, lane + t * 16, _BIG)))
            g = jnp.minimum(g, _NB - 1)
            nv = jnp.minimum(_BLK, _NV - g * _BLK)

            def scan_pos(k, acc):
                v = sc_v[pl.ds((g * _BLK + k) * 16, 16)]
                return jnp.minimum(
                    acc, jnp.where(v == s, lane + k * 16, _BIG))

            posv = lax.fori_loop(0, nv, scan_pos, _splat_i(_BIG))
            pos = jnp.minimum(jnp.min(posv), _BLK * 16 - 1)
            kv = pos // 16
            pl0 = pos - kv * 16
            i0 = g * _BLK + kv
            gi = i0 * 16 + pl0
            v0 = sc_v[pl.ds(i0 * 16, 16)]
            sc_v[pl.ds(i0 * 16, 16)] = jnp.where(lane == pl0, _NEG, v0)

            def rescan(k, acc):
                return jnp.maximum(acc, sc_v[pl.ds((g * _BLK + k) * 16, 16)])

            bm = jnp.max(lax.fori_loop(0, nv, rescan, _splat_f(_NEG)))
            jb = (g // 16) * 16
            l1_v[pl.ds(jb, 16)] = jnp.where(
                lane == (g - jb), bm, l1_v[pl.ds(jb, 16)])
            gis = _splat_i(0) + gi
            bx1 = plsc.load_gather(x1_v, [gis])
            by1 = plsc.load_gather(y1_v, [gis])
            bx2 = plsc.load_gather(x2_v, [gis])
            by2 = plsc.load_gather(y2_v, [gis])
            barea = (bx2 - bx1) * (by2 - by1)

            def chk(t, anyov):
                qx1 = ax1_v[pl.ds(t * 16, 16)]
                qy1 = ay1_v[pl.ds(t * 16, 16)]
                qx2 = ax2_v[pl.ds(t * 16, 16)]
                qy2 = ay2_v[pl.ds(t * 16, 16)]
                xx1 = jnp.maximum(qx1, bx1)
                yy1 = jnp.maximum(qy1, by1)
                xx2 = jnp.minimum(qx2, bx2)
                yy2 = jnp.minimum(qy2, by2)
                inter = (jnp.maximum(xx2 - xx1, 0.0)
                         * jnp.maximum(yy2 - yy1, 0.0))
                qarea = (qx2 - qx1) * (qy2 - qy1)
                iou = inter / (qarea + barea - inter + 1e-8)
                return anyov | (iou > _NMS_T)

            nacc = (count + 15) // 16
            anyov = lax.fori_loop(0, nacc, chk,
                                  jnp.zeros((16,), dtype=jnp.bool_))
            accept = valid & jnp.logical_not(jnp.any(anyov))
            base = (count // 16) * 16
            wm = (lane == (count - base)) & accept
            ax1_v[pl.ds(base, 16)] = jnp.where(wm, bx1, ax1_v[pl.ds(base, 16)])
            ay1_v[pl.ds(base, 16)] = jnp.where(wm, by1, ay1_v[pl.ds(base, 16)])
            ax2_v[pl.ds(base, 16)] = jnp.where(wm, bx2, ax2_v[pl.ds(base, 16)])
            ay2_v[pl.ds(base, 16)] = jnp.where(wm, by2, ay2_v[pl.ds(base, 16)])
            osc_v[pl.ds(base, 16)] = jnp.where(
                wm, _splat_f(0.0) + s, osc_v[pl.ds(base, 16)])
            oidx_v[pl.ds(base, 16)] = jnp.where(
                wm, gis, oidx_v[pl.ds(base, 16)])
            count = count + jnp.where(accept, 1, 0)
            return count, jnp.logical_not(valid)

        lax.while_loop(greedy_cond, greedy_body, (jnp.int32(0), jnp.bool_(False)))

        pltpu.sync_copy(osc_v, sh_sc.at[pl.ds(sub * _SLOTS, _SLOTS)])
        pltpu.sync_copy(oidx_v, sh_idx.at[pl.ds(sub * _SLOTS, _SLOTS)])

    plsc.subcore_barrier()

    @pl.when((core == 0) & (sub == 0))
    def _merge():
        pltpu.sync_copy(sh_sc, msc_v)
        pltpu.sync_copy(sh_idx, midx_v)
        cbase = jnp.where(lane < _NC, lane * _SLOTS, 0)

        def init_out(t, _):
            sout_v[pl.ds(t * 16, 16)] = _splat_f(-1.0)
            lout_v[pl.ds(t * 16, 16)] = _splat_i(-1)
            gidx_v[pl.ds(t * 16, 16)] = _splat_i(0)
            return 0

        lax.fori_loop(0, _SLOTS // 16, init_out, 0)

        def merge_step(j, p):
            addr = cbase + jnp.minimum(p, _SLOTS - 1)
            h = plsc.load_gather(msc_v, [addr])
            h = jnp.where(lane < _NC, h, _NEG)
            m = jnp.max(h)
            valid = m > _NEG / 2
            bl = jnp.min(jnp.where(h == m, lane, _BIG))
            bl = jnp.minimum(bl, _NC - 1)
            gidx16 = plsc.load_gather(midx_v, [addr])
            gi = jnp.max(jnp.where(lane == bl, gidx16, -1))
            base = (j // 16) * 16
            wm = lane == (j - base)
            sout_v[pl.ds(base, 16)] = jnp.where(
                wm, jnp.where(valid, _splat_f(0.0) + m, -1.0),
                sout_v[pl.ds(base, 16)])
            lout_v[pl.ds(base, 16)] = jnp.where(
                wm, jnp.where(valid, _splat_i(0) + bl, -1),
                lout_v[pl.ds(base, 16)])
            gidx_v[pl.ds(base, 16)] = jnp.where(
                wm, jnp.where(valid, jnp.maximum(_splat_i(0) + gi, 0), 0),
                gidx_v[pl.ds(base, 16)])
            return p + jnp.where((lane == bl) & valid, 1, 0)

        lax.fori_loop(0, _MD, merge_step, _splat_i(0))

        srcs = (x1_h, y1_h, x2_h, y2_h, t0_h, t1_h, t2_h, r0_h, r1_h, r2_h)
        copies = [
            pltpu.async_copy(src.at[gidx_v],
                             big_v.at[pl.ds(f * _SLOTS, _SLOTS)], dsem)
            for f, src in enumerate(srcs)
        ]
        for cp in copies:
            cp.wait()

        def mask_fields(t, _):
            vmask = sout_v[pl.ds(t * 16, 16)] >= 0.0
            for f in range(10):
                o = f * _SLOTS + t * 16
                big_v[pl.ds(o, 16)] = jnp.where(
                    vmask, big_v[pl.ds(o, 16)], -1.0)
            return 0

        lax.fori_loop(0, _SLOTS // 16, mask_fields, 0)
        pltpu.sync_copy(big_v, of_h)
        pltpu.sync_copy(sout_v, os_h)
        pltpu.sync_copy(lout_v, ol_h)


_mesh = plsc.VectorSubcoreMesh(core_axis_name="c", subcore_axis_name="s")

_sc_call = functools.partial(
    pl.kernel,
    mesh=_mesh,
    compiler_params=pltpu.CompilerParams(needs_layout_passes=False),
    out_type=[
        jax.ShapeDtypeStruct((10 * _SLOTS,), jnp.float32),
        jax.ShapeDtypeStruct((_SLOTS,), jnp.float32),
        jax.ShapeDtypeStruct((_SLOTS,), jnp.int32),
    ],
    scratch_types=[
        pltpu.VMEM((_N,), jnp.float32),
        pltpu.VMEM((_N,), jnp.float32),
        pltpu.VMEM((_N,), jnp.float32),
        pltpu.VMEM((_N,), jnp.float32),
        pltpu.VMEM((_N,), jnp.float32),
        pltpu.VMEM((_NL1,), jnp.float32),
        pltpu.VMEM((_SLOTS,), jnp.float32),
        pltpu.VMEM((_SLOTS,), jnp.float32),
        pltpu.VMEM((_SLOTS,), jnp.float32),
        pltpu.VMEM((_SLOTS,), jnp.float32),
        pltpu.VMEM((_SLOTS,), jnp.float32),
        pltpu.VMEM((_SLOTS,), jnp.int32),
        pltpu.VMEM_SHARED((_NC * _SLOTS,), jnp.float32),
        pltpu.VMEM_SHARED((_NC * _SLOTS,), jnp.int32),
        pltpu.VMEM((_NC * _SLOTS,), jnp.float32),
        pltpu.VMEM((_NC * _SLOTS,), jnp.int32),
        pltpu.VMEM((_SLOTS,), jnp.float32),
        pltpu.VMEM((_SLOTS,), jnp.int32),
        pltpu.VMEM((_SLOTS,), jnp.int32),
        pltpu.VMEM((10 * _SLOTS,), jnp.float32),
        pltpu.SemaphoreType.DMA,
        pltpu.SemaphoreType.DMA,
    ],
)(_sc_body)


def kernel(boxes, classification, translation, rotation):
    b = boxes[0]
    c = classification[0]
    t = translation[0]
    r = rotation[0]

    scores = c.T.reshape(-1)
    of, os, ol = _sc_call(scores, b[:, 0], b[:, 1], b[:, 2], b[:, 3],
                          t[:, 0], t[:, 1], t[:, 2],
                          r[:, 0], r[:, 1], r[:, 2])

    m = of.reshape(10, _SLOTS)
    out_b = m[0:4, :_MD].T
    out_t = m[4:7, :_MD].T
    out_r = m[7:10, :_MD].T
    out_s = os[:_MD]
    out_l = ol[:_MD]
    return (out_b[None], out_s[None], out_l[None], out_t[None], out_r[None])


# speedup vs baseline: 1.2776x; 1.2776x over previous
import functools

import jax
import jax.numpy as jnp
from jax import lax
from jax.experimental import pallas as pl
from jax.experimental.pallas import tpu as pltpu
from jax.experimental.pallas import tpu_sc as plsc

_NC = 8
_N = 20000
_NV = _N // 16
_BLK = 16
_NB = (_NV + _BLK - 1) // _BLK
_NVP = _NB * _BLK
_NL1 = 80
_MD = 100
_SLOTS = 112
_NMS_T = 0.5
_SCORE_T = 0.01
_NEG = -1e9
_BIG = 2**30
_SENT = 2e9


def _lane():
    return lax.iota(jnp.int32, 16)


def _splat_f(x):
    return jnp.full((16,), x, dtype=jnp.float32)


def _splat_i(x):
    return jnp.full((16,), x, dtype=jnp.int32)


def _sc_body(scores_h, x1_h, y1_h, x2_h, y2_h,
             t0_h, t1_h, t2_h, r0_h, r1_h, r2_h,
             of_h, os_h, ol_h,
             sc_v, x1_v, y1_v, x2_v, y2_v, l1_v,
             ax1_v, ay1_v, ax2_v, ay2_v, osc_v, oidx_v,
             sh_sc, sh_idx, msc_v, midx_v, sout_v, lout_v, gidx_v, big_v,
             dsem, bsem):
    core = lax.axis_index("c")
    sub = lax.axis_index("s")
    lane = _lane()
    active = (core == 0) & (sub < _NC)

    @pl.when(active)
    def _per_class():
        bcopies = [
            pltpu.async_copy(x1_h, x1_v, bsem),
            pltpu.async_copy(y1_h, y1_v, bsem),
            pltpu.async_copy(x2_h, x2_v, bsem),
            pltpu.async_copy(y2_h, y2_v, bsem),
        ]
        pltpu.sync_copy(scores_h.at[pl.ds(sub * _N, _N)], sc_v.at[pl.ds(0, _N)])

        def pad_tail(i, _):
            sc_v[pl.ds(i * 16, 16)] = _splat_f(_NEG)
            return 0

        lax.fori_loop(_NV, _NVP, pad_tail, 0)

        def build_block(g, _):
            acc = _splat_f(_NEG)
            for k in range(_BLK):
                v = sc_v[pl.ds((g * _BLK + k) * 16, 16)]
                v = jnp.where(v >= _SCORE_T, v, _NEG)
                sc_v[pl.ds((g * _BLK + k) * 16, 16)] = v
                acc = jnp.maximum(acc, v)
            m = jnp.max(acc)
            jb = (g // 16) * 16
            l1_v[pl.ds(jb, 16)] = jnp.where(
                lane == (g - jb), m, l1_v[pl.ds(jb, 16)])
            return 0

        l1_v[pl.ds(4 * 16, 16)] = _splat_f(_NEG)
        lax.fori_loop(0, _NB, build_block, 0)

        def init_slots(t, _):
            ax1_v[pl.ds(t * 16, 16)] = _splat_f(_SENT)
            ay1_v[pl.ds(t * 16, 16)] = _splat_f(_SENT)
            ax2_v[pl.ds(t * 16, 16)] = _splat_f(_SENT)
            ay2_v[pl.ds(t * 16, 16)] = _splat_f(_SENT)
            osc_v[pl.ds(t * 16, 16)] = _splat_f(_NEG)
            oidx_v[pl.ds(t * 16, 16)] = _splat_i(-1)
            return 0

        lax.fori_loop(0, _SLOTS // 16, init_slots, 0)
        for cp in bcopies:
            cp.wait()

        def greedy_cond(state):
            count, done = state
            return (count < _MD) & jnp.logical_not(done)

        def greedy_body(state):
            count, done = state
            m1 = _splat_f(_NEG)
            for t in range(_NL1 // 16):
                m1 = jnp.maximum(m1, l1_v[pl.ds(t * 16, 16)])
            s = jnp.max(m1)
            valid = s > _NEG / 2
            g = _BIG
            for t in range(_NL1 // 16):
                v1 = l1_v[pl.ds(t * 16, 16)]
                g = jnp.minimum(
                    g, jnp.min(jnp.where(v1 == s, lane + t * 16, _BIG)))
            g = jnp.minimum(g, _NB - 1)
            posv = _splat_i(_BIG)
            for k in range(_BLK):
                v = sc_v[pl.ds((g * _BLK + k) * 16, 16)]
                posv = jnp.minimum(
                    posv, jnp.where(v == s, lane + k * 16, _BIG))
            pos = jnp.minimum(jnp.min(posv), _BLK * 16 - 1)
            kv = pos // 16
            pl0 = pos - kv * 16
            i0 = g * _BLK + kv
            gi = i0 * 16 + pl0
            v0 = sc_v[pl.ds(i0 * 16, 16)]
            sc_v[pl.ds(i0 * 16, 16)] = jnp.where(lane == pl0, _NEG, v0)

            racc = _splat_f(_NEG)
            for k in range(_BLK):
                racc = jnp.maximum(racc, sc_v[pl.ds((g * _BLK + k) * 16, 16)])
            bm = jnp.max(racc)
            jb = (g // 16) * 16
            l1_v[pl.ds(jb, 16)] = jnp.where(
                lane == (g - jb), bm, l1_v[pl.ds(jb, 16)])
            gis = _splat_i(0) + gi
            bx1 = plsc.load_gather(x1_v, [gis])
            by1 = plsc.load_gather(y1_v, [gis])
            bx2 = plsc.load_gather(x2_v, [gis])
            by2 = plsc.load_gather(y2_v, [gis])
            barea = (bx2 - bx1) * (by2 - by1)

            def chk(t, anyov):
                qx1 = ax1_v[pl.ds(t * 16, 16)]
                qy1 = ay1_v[pl.ds(t * 16, 16)]
                qx2 = ax2_v[pl.ds(t * 16, 16)]
                qy2 = ay2_v[pl.ds(t * 16, 16)]
                xx1 = jnp.maximum(qx1, bx1)
                yy1 = jnp.maximum(qy1, by1)
                xx2 = jnp.minimum(qx2, bx2)
                yy2 = jnp.minimum(qy2, by2)
                inter = (jnp.maximum(xx2 - xx1, 0.0)
                         * jnp.maximum(yy2 - yy1, 0.0))
                qarea = (qx2 - qx1) * (qy2 - qy1)
                iou = inter / (qarea + barea - inter + 1e-8)
                return anyov | (iou > _NMS_T)

            anyov = jnp.zeros((16,), dtype=jnp.bool_)
            for t in range(_SLOTS // 16):
                anyov = chk(t, anyov)
            accept = valid & jnp.logical_not(jnp.any(anyov))
            base = (count // 16) * 16
            wm = (lane == (count - base)) & accept
            ax1_v[pl.ds(base, 16)] = jnp.where(wm, bx1, ax1_v[pl.ds(base, 16)])
            ay1_v[pl.ds(base, 16)] = jnp.where(wm, by1, ay1_v[pl.ds(base, 16)])
            ax2_v[pl.ds(base, 16)] = jnp.where(wm, bx2, ax2_v[pl.ds(base, 16)])
            ay2_v[pl.ds(base, 16)] = jnp.where(wm, by2, ay2_v[pl.ds(base, 16)])
            osc_v[pl.ds(base, 16)] = jnp.where(
                wm, _splat_f(0.0) + s, osc_v[pl.ds(base, 16)])
            oidx_v[pl.ds(base, 16)] = jnp.where(
                wm, gis, oidx_v[pl.ds(base, 16)])
            count = count + jnp.where(accept, 1, 0)
            return count, jnp.logical_not(valid)

        lax.while_loop(greedy_cond, greedy_body, (jnp.int32(0), jnp.bool_(False)))

        pltpu.sync_copy(osc_v, sh_sc.at[pl.ds(sub * _SLOTS, _SLOTS)])
        pltpu.sync_copy(oidx_v, sh_idx.at[pl.ds(sub * _SLOTS, _SLOTS)])

    plsc.subcore_barrier()

    @pl.when((core == 0) & (sub == 0))
    def _merge():
        pltpu.sync_copy(sh_sc, msc_v)
        pltpu.sync_copy(sh_idx, midx_v)
        cbase = jnp.where(lane < _NC, lane * _SLOTS, 0)

        def init_out(t, _):
            sout_v[pl.ds(t * 16, 16)] = _splat_f(-1.0)
            lout_v[pl.ds(t * 16, 16)] = _splat_i(-1)
            gidx_v[pl.ds(t * 16, 16)] = _splat_i(0)
            return 0

        lax.fori_loop(0, _SLOTS // 16, init_out, 0)

        def merge_step(j, p):
            addr = cbase + jnp.minimum(p, _SLOTS - 1)
            h = plsc.load_gather(msc_v, [addr])
            h = jnp.where(lane < _NC, h, _NEG)
            m = jnp.max(h)
            valid = m > _NEG / 2
            bl = jnp.min(jnp.where(h == m, lane, _BIG))
            bl = jnp.minimum(bl, _NC - 1)
            gidx16 = plsc.load_gather(midx_v, [addr])
            gi = jnp.max(jnp.where(lane == bl, gidx16, -1))
            base = (j // 16) * 16
            wm = lane == (j - base)
            sout_v[pl.ds(base, 16)] = jnp.where(
                wm, jnp.where(valid, _splat_f(0.0) + m, -1.0),
                sout_v[pl.ds(base, 16)])
            lout_v[pl.ds(base, 16)] = jnp.where(
                wm, jnp.where(valid, _splat_i(0) + bl, -1),
                lout_v[pl.ds(base, 16)])
            gidx_v[pl.ds(base, 16)] = jnp.where(
                wm, jnp.where(valid, jnp.maximum(_splat_i(0) + gi, 0), 0),
                gidx_v[pl.ds(base, 16)])
            return p + jnp.where((lane == bl) & valid, 1, 0)

        lax.fori_loop(0, _MD, merge_step, _splat_i(0))

        srcs = (x1_h, y1_h, x2_h, y2_h, t0_h, t1_h, t2_h, r0_h, r1_h, r2_h)
        copies = [
            pltpu.async_copy(src.at[gidx_v],
                             big_v.at[pl.ds(f * _SLOTS, _SLOTS)], dsem)
            for f, src in enumerate(srcs)
        ]
        for cp in copies:
            cp.wait()

        def mask_fields(t, _):
            vmask = sout_v[pl.ds(t * 16, 16)] >= 0.0
            for f in range(10):
                o = f * _SLOTS + t * 16
                big_v[pl.ds(o, 16)] = jnp.where(
                    vmask, big_v[pl.ds(o, 16)], -1.0)
            return 0

        lax.fori_loop(0, _SLOTS // 16, mask_fields, 0)
        pltpu.sync_copy(big_v, of_h)
        pltpu.sync_copy(sout_v, os_h)
        pltpu.sync_copy(lout_v, ol_h)


_mesh = plsc.VectorSubcoreMesh(core_axis_name="c", subcore_axis_name="s")

_sc_call = functools.partial(
    pl.kernel,
    mesh=_mesh,
    compiler_params=pltpu.CompilerParams(needs_layout_passes=False),
    out_type=[
        jax.ShapeDtypeStruct((10 * _SLOTS,), jnp.float32),
        jax.ShapeDtypeStruct((_SLOTS,), jnp.float32),
        jax.ShapeDtypeStruct((_SLOTS,), jnp.int32),
    ],
    scratch_types=[
        pltpu.VMEM((_NVP * 16,), jnp.float32),
        pltpu.VMEM((_N,), jnp.float32),
        pltpu.VMEM((_N,), jnp.float32),
        pltpu.VMEM((_N,), jnp.float32),
        pltpu.VMEM((_N,), jnp.float32),
        pltpu.VMEM((_NL1,), jnp.float32),
        pltpu.VMEM((_SLOTS,), jnp.float32),
        pltpu.VMEM((_SLOTS,), jnp.float32),
        pltpu.VMEM((_SLOTS,), jnp.float32),
        pltpu.VMEM((_SLOTS,), jnp.float32),
        pltpu.VMEM((_SLOTS,), jnp.float32),
        pltpu.VMEM((_SLOTS,), jnp.int32),
        pltpu.VMEM_SHARED((_NC * _SLOTS,), jnp.float32),
        pltpu.VMEM_SHARED((_NC * _SLOTS,), jnp.int32),
        pltpu.VMEM((_NC * _SLOTS,), jnp.float32),
        pltpu.VMEM((_NC * _SLOTS,), jnp.int32),
        pltpu.VMEM((_SLOTS,), jnp.float32),
        pltpu.VMEM((_SLOTS,), jnp.int32),
        pltpu.VMEM((_SLOTS,), jnp.int32),
        pltpu.VMEM((10 * _SLOTS,), jnp.float32),
        pltpu.SemaphoreType.DMA,
        pltpu.SemaphoreType.DMA,
    ],
)(_sc_body)


def kernel(boxes, classification, translation, rotation):
    b = boxes[0]
    c = classification[0]
    t = translation[0]
    r = rotation[0]

    scores = c.T.reshape(-1)
    of, os, ol = _sc_call(scores, b[:, 0], b[:, 1], b[:, 2], b[:, 3],
                          t[:, 0], t[:, 1], t[:, 2],
                          r[:, 0], r[:, 1], r[:, 2])

    m = of.reshape(10, _SLOTS)
    out_b = m[0:4, :_MD].T
    out_t = m[4:7, :_MD].T
    out_r = m[7:10, :_MD].T
    out_s = os[:_MD]
    out_l = ol[:_MD]
    return (out_b[None], out_s[None], out_l[None], out_t[None], out_r[None])
